# Initial kernel scaffold; baseline (speedup 1.0000x reference)
import jax
import jax.numpy as jnp
from jax.experimental import pallas as pl


def kernel(x, edge_attr, edge_index, batch, W_node, b_node, W_eenc, b_eenc, Wm, bm, Wm2, bm2, Wu, bu, We, be, Wep, bep, Wr1, br1, Wr2, br2):
    raise NotImplementedError("write your pallas kernel here")


# speedup vs baseline: 1.2099x; 1.2099x over previous
import functools

import jax
import jax.numpy as jnp
from jax.experimental import pallas as pl
from jax.experimental.pallas import tpu as pltpu

F32 = jnp.float32
G_GRAPHS = 128


def _relu(v):
    return jnp.maximum(v, 0.0)


def _dot(a, b):
    return jax.lax.dot_general(a, b, (((1,), (0,)), ((), ())),
                               preferred_element_type=F32)


def _pick_block(total, cands):
    for c in cands:
        if total % c == 0:
            return c
    return total


def _enc_node_body(x_ref, w_ref, b_ref, batch_ref, h_ref, starts_ref):
    h_ref[...] = _relu(_dot(x_ref[...], w_ref[...]) + b_ref[...])
    lane = jax.lax.broadcasted_iota(jnp.int32, (1, G_GRAPHS), 1).astype(F32)
    lt = (batch_ref[...] < lane).astype(F32)
    part = jnp.sum(lt, axis=0, keepdims=True)

    @pl.when(pl.program_id(0) == 0)
    def _():
        starts_ref[...] = jnp.zeros_like(starts_ref)

    starts_ref[...] += part


def _enc_node(x, w, b, batch_f):
    n, din = x.shape
    h_dim = w.shape[1]
    b = b.reshape(1, -1)
    nb = _pick_block(n, [2000, 1000, 500, 200, 100, 50, 10])
    return pl.pallas_call(
        _enc_node_body,
        grid=(n // nb,),
        in_specs=[
            pl.BlockSpec((nb, din), lambda i: (i, 0)),
            pl.BlockSpec((din, h_dim), lambda i: (0, 0)),
            pl.BlockSpec((1, h_dim), lambda i: (0, 0)),
            pl.BlockSpec((nb, 1), lambda i: (i, 0)),
        ],
        out_specs=[
            pl.BlockSpec((nb, h_dim), lambda i: (i, 0)),
            pl.BlockSpec((1, G_GRAPHS), lambda i: (0, 0)),
        ],
        out_shape=[
            jax.ShapeDtypeStruct((n, h_dim), F32),
            jax.ShapeDtypeStruct((1, G_GRAPHS), F32),
        ],
    )(x, w, b, batch_f)


def _edge_layer_body(hs_ref, hd_ref, e_ref, wms, wmd, wme, bm, wm2, bm2,
                     wes, wed, wee, be, m_ref, en_ref, *, enc=None):
    hs = hs_ref[...]
    hd = hd_ref[...]
    e = e_ref[...]
    if enc is not None:
        wenc, benc = enc
        e = _relu(_dot(e, wenc[...]) + benc[...])
    mp = _relu(_dot(hs, wms[...]) + _dot(hd, wmd[...]) + _dot(e, wme[...])
               + bm[...])
    m_ref[...] = _dot(mp, wm2[...]) + bm2[...]
    en_ref[...] = _relu(_dot(hs, wes[...]) + _dot(hd, wed[...])
                        + _dot(e, wee[...]) + be[...])


def _edge_layer0_body(hs_ref, hd_ref, ea_ref, wenc, benc, wms, wmd, wme, bm,
                      wm2, bm2, wes, wed, wee, be, m_ref, en_ref):
    _edge_layer_body(hs_ref, hd_ref, ea_ref, wms, wmd, wme, bm, wm2, bm2,
                     wes, wed, wee, be, m_ref, en_ref, enc=(wenc, benc))


def _full(shape):
    return pl.BlockSpec(shape, lambda i: tuple(0 for _ in shape))


def _edge_layer(hs, hd, e, wenc, benc, wm, bm, wm2, bm2, we, be):
    e_edges, h_dim = hs.shape
    de = e.shape[1]
    hm = wm.shape[1]
    eb = _pick_block(e_edges, [3200, 2560, 2048, 1600, 1280, 1024, 800, 640,
                               512, 400, 320, 256, 200, 160, 128, 80, 64])
    wms, wmd, wme = wm[:h_dim], wm[h_dim:2 * h_dim], wm[2 * h_dim:]
    wes, wed, wee = we[:h_dim], we[h_dim:2 * h_dim], we[2 * h_dim:]
    row = lambda i: (i, 0)
    in_specs = [pl.BlockSpec((eb, h_dim), row), pl.BlockSpec((eb, h_dim), row),
                pl.BlockSpec((eb, de), row)]
    args = [hs, hd, e]
    if wenc is not None:
        body = _edge_layer0_body
        in_specs += [_full(wenc.shape), _full((1, h_dim))]
        args += [wenc, benc.reshape(1, -1)]
    else:
        body = _edge_layer_body
    in_specs += [_full(wms.shape), _full(wmd.shape), _full(wme.shape),
                 _full((1, hm)), _full(wm2.shape), _full((1, h_dim)),
                 _full(wes.shape), _full(wed.shape), _full(wee.shape),
                 _full((1, h_dim))]
    args += [wms, wmd, wme, bm.reshape(1, -1), wm2, bm2.reshape(1, -1),
             wes, wed, wee, be.reshape(1, -1)]
    return pl.pallas_call(
        body,
        grid=(e_edges // eb,),
        in_specs=in_specs,
        out_specs=[pl.BlockSpec((eb, h_dim), row),
                   pl.BlockSpec((eb, h_dim), row)],
        out_shape=[jax.ShapeDtypeStruct((e_edges, h_dim), F32),
                   jax.ShapeDtypeStruct((e_edges, h_dim), F32)],
    )(*args)


def _edge_last_body(hs_ref, hd_ref, e_ref, wes, wed, wee, be, wep, bep,
                    wr1, br1, wr2, br2, srcf_ref, starts_ref, g_ref):
    en = _relu(_dot(hs_ref[...], wes[...]) + _dot(hd_ref[...], wed[...])
               + _dot(e_ref[...], wee[...]) + be[...])
    e20 = _dot(en, wep[...]) + bep[...]
    r = _dot(_relu(_dot(e20, wr1[...]) + br1[...]), wr2[...]) + br2[...]
    srcv = srcf_ref[0]
    ge = (srcv >= starts_ref[...]).astype(F32)
    gid = jnp.sum(ge, axis=1, keepdims=True)
    lane = jax.lax.broadcasted_iota(jnp.int32, srcv.shape[:1] + (G_GRAPHS,),
                                    1).astype(F32) + 1.0
    onehot = (gid == lane).astype(F32)
    gpart = jax.lax.dot_general(onehot, r, (((0,), (0,)), ((), ())),
                                preferred_element_type=F32)

    @pl.when(pl.program_id(0) == 0)
    def _():
        g_ref[...] = jnp.zeros_like(g_ref)

    g_ref[...] += gpart


def _edge_last(hs, hd, e, we, be, wep, bep, wr1, br1, wr2, br2, srcf, starts):
    e_edges, h_dim = hs.shape
    dout = wr2.shape[1]
    eb = _pick_block(e_edges, [3200, 2560, 2048, 1600, 1280, 1024, 800, 640,
                               512, 400, 320, 256, 200, 160, 128, 80, 64])
    wes, wed, wee = we[:h_dim], we[h_dim:2 * h_dim], we[2 * h_dim:]
    row = lambda i: (i, 0)
    srcf3 = srcf.reshape(e_edges // eb, eb, 1)
    return pl.pallas_call(
        _edge_last_body,
        grid=(e_edges // eb,),
        in_specs=[
            pl.BlockSpec((eb, h_dim), row), pl.BlockSpec((eb, h_dim), row),
            pl.BlockSpec((eb, h_dim), row),
            _full(wes.shape), _full(wed.shape), _full(wee.shape),
            _full((1, h_dim)),
            _full(wep.shape), _full((1, wep.shape[1])),
            _full(wr1.shape), _full((1, wr1.shape[1])),
            _full(wr2.shape), _full((1, dout)),
            pl.BlockSpec((1, eb, 1), lambda i: (i, 0, 0)),
            _full((1, G_GRAPHS)),
        ],
        out_specs=pl.BlockSpec((G_GRAPHS, dout), lambda i: (0, 0)),
        out_shape=jax.ShapeDtypeStruct((G_GRAPHS, dout), F32),
    )(hs, hd, e, wes, wed, wee, be.reshape(1, -1), wep, bep.reshape(1, -1),
      wr1, br1.reshape(1, -1), wr2, br2.reshape(1, -1), srcf3, starts)


def _update_body(h_ref, a_ref, wuh, wua, bu, o_ref):
    o_ref[...] = _relu(_dot(h_ref[...], wuh[...]) + _dot(a_ref[...], wua[...])
                       + bu[...])


def _update(h, agg, wu, bu):
    n, h_dim = h.shape
    nb = _pick_block(n, [2000, 1000, 500, 200, 100, 50, 10])
    wuh, wua = wu[:h_dim], wu[h_dim:]
    row = lambda i: (i, 0)
    return pl.pallas_call(
        _update_body,
        grid=(n // nb,),
        in_specs=[pl.BlockSpec((nb, h_dim), row), pl.BlockSpec((nb, h_dim), row),
                  _full(wuh.shape), _full(wua.shape), _full((1, h_dim))],
        out_specs=pl.BlockSpec((nb, h_dim), row),
        out_shape=jax.ShapeDtypeStruct((n, h_dim), F32),
    )(h, agg, wuh, wua, bu.reshape(1, -1))


def _gather_rows(h, src, dst):
    return jnp.take(h, src, axis=0), jnp.take(h, dst, axis=0)


def _segment_sum(m, dst, n):
    return jax.ops.segment_sum(m, dst, num_segments=n)


def kernel(x, edge_attr, edge_index, batch, W_node, b_node, W_eenc, b_eenc,
           Wm, bm, Wm2, bm2, Wu, bu, We, be, Wep, bep, Wr1, br1, Wr2, br2):
    n = x.shape[0]
    src = edge_index[0]
    dst = edge_index[1]
    batch_f = batch.astype(F32).reshape(n, 1)
    src_f = src.astype(F32)

    h, starts = _enc_node(x, W_node, b_node, batch_f)

    hs, hd = _gather_rows(h, src, dst)
    m, e = _edge_layer(hs, hd, edge_attr, W_eenc, b_eenc,
                       Wm[0], bm[0], Wm2[0], bm2[0], We[0], be[0])
    agg = _segment_sum(m, dst, n)
    h = _update(h, agg, Wu[0], bu[0])

    hs, hd = _gather_rows(h, src, dst)
    m, e = _edge_layer(hs, hd, e, None, None,
                       Wm[1], bm[1], Wm2[1], bm2[1], We[1], be[1])
    agg = _segment_sum(m, dst, n)
    h = _update(h, agg, Wu[1], bu[1])

    hs, hd = _gather_rows(h, src, dst)
    g = _edge_last(hs, hd, e, We[2], be[2], Wep, bep, Wr1, br1, Wr2, br2,
                   src_f, starts)
    return g


# speedup vs baseline: 3.5307x; 2.9182x over previous
import functools

import jax
import jax.numpy as jnp
from jax import lax
from jax.experimental import pallas as pl
from jax.experimental.pallas import tpu as pltpu
from jax.experimental.pallas import tpu_sc as plsc

F32 = jnp.float32
G_GRAPHS = 128
HP = 128


def _relu(v):
    return jnp.maximum(v, 0.0)


def _dot(a, b):
    return jax.lax.dot_general(a, b, (((1,), (0,)), ((), ())),
                               preferred_element_type=F32)


def _pick_block(total, cands):
    for c in cands:
        if total % c == 0:
            return c
    return total


def _padc(w, cols):
    return jnp.pad(w, ((0, 0), (0, cols - w.shape[1])))


def _padr(w, rows):
    return jnp.pad(w, ((0, rows - w.shape[0]), (0, 0)))


def _full(shape):
    return pl.BlockSpec(shape, lambda i: tuple(0 for _ in shape))


_EDGE_CANDS = [3200, 2560, 2048, 1600, 1280, 1024, 800, 640, 512, 400, 320,
               256, 200, 160, 128, 80, 64]
_NODE_CANDS = [2000, 1000, 500, 200, 100, 50, 10]


def _enc_node_body(x_ref, w_ref, b_ref, batch_ref, h_ref, starts_ref):
    h_ref[...] = _relu(_dot(x_ref[...], w_ref[...]) + b_ref[...])
    lane = jax.lax.broadcasted_iota(jnp.int32, (1, G_GRAPHS), 1).astype(F32)
    lt = (batch_ref[...] < lane).astype(F32)
    part = jnp.sum(lt, axis=0, keepdims=True)

    @pl.when(pl.program_id(0) == 0)
    def _():
        starts_ref[...] = jnp.zeros_like(starts_ref)

    starts_ref[...] += part


def _enc_node(x, w, b, batch_f):
    n, din = x.shape
    nb = _pick_block(n, _NODE_CANDS)
    return pl.pallas_call(
        _enc_node_body,
        grid=(n // nb,),
        in_specs=[
            pl.BlockSpec((nb, din), lambda i: (i, 0)),
            pl.BlockSpec((din, HP), lambda i: (0, 0)),
            pl.BlockSpec((1, HP), lambda i: (0, 0)),
            pl.BlockSpec((nb, 1), lambda i: (i, 0)),
        ],
        out_specs=[
            pl.BlockSpec((nb, HP), lambda i: (i, 0)),
            pl.BlockSpec((1, G_GRAPHS), lambda i: (0, 0)),
        ],
        out_shape=[
            jax.ShapeDtypeStruct((n, HP), F32),
            jax.ShapeDtypeStruct((1, G_GRAPHS), F32),
        ],
    )(x, w, b, batch_f)


def _edge_layer_body(hs_ref, hd_ref, e_ref, wms, wmd, wme, bm, wm2, bm2,
                     wes, wed, wee, be, m_ref, en_ref, *, enc=None):
    hs = hs_ref[...]
    hd = hd_ref[...]
    e = e_ref[...]
    if enc is not None:
        wenc, benc = enc
        e = _relu(_dot(e, wenc[...]) + benc[...])
    mp = _relu(_dot(hs, wms[...]) + _dot(hd, wmd[...]) + _dot(e, wme[...])
               + bm[...])
    m_ref[...] = _dot(mp, wm2[...]) + bm2[...]
    en_ref[...] = _relu(_dot(hs, wes[...]) + _dot(hd, wed[...])
                        + _dot(e, wee[...]) + be[...])


def _edge_layer0_body(hs_ref, hd_ref, ea_ref, wenc, benc, wms, wmd, wme, bm,
                      wm2, bm2, wes, wed, wee, be, m_ref, en_ref):
    _edge_layer_body(hs_ref, hd_ref, ea_ref, wms, wmd, wme, bm, wm2, bm2,
                     wes, wed, wee, be, m_ref, en_ref, enc=(wenc, benc))


def _edge_layer(gat, e, wenc, benc, wms, wmd, wme, bm, wm2, bm2,
                wes, wed, wee, be):
    e_edges = gat.shape[0] // 2
    m_dim = wm2.shape[1]
    en_dim = wee.shape[1]
    de = e.shape[1]
    eb = _pick_block(e_edges, _EDGE_CANDS)
    off = e_edges // eb
    row = lambda i: (i, 0)
    in_specs = [pl.BlockSpec((eb, HP), row),
                pl.BlockSpec((eb, HP), lambda i: (i + off, 0)),
                pl.BlockSpec((eb, de), row)]
    args = [gat, gat, e]
    if wenc is not None:
        body = _edge_layer0_body
        in_specs += [_full(wenc.shape), _full(benc.shape)]
        args += [wenc, benc]
    else:
        body = _edge_layer_body
    for wref in (wms, wmd, wme, bm, wm2, bm2, wes, wed, wee, be):
        in_specs.append(_full(wref.shape))
        args.append(wref)
    return pl.pallas_call(
        body,
        grid=(e_edges // eb,),
        in_specs=in_specs,
        out_specs=[pl.BlockSpec((eb, m_dim), row),
                   pl.BlockSpec((eb, en_dim), row)],
        out_shape=[jax.ShapeDtypeStruct((e_edges, m_dim), F32),
                   jax.ShapeDtypeStruct((e_edges, en_dim), F32)],
    )(*args)


def _edge_last_body(hs_ref, hd_ref, e_ref, wes, wed, wee, be, wep, bep,
                    wr1, br1, wr2, br2, srcf_ref, starts_ref, g_ref):
    en = _relu(_dot(hs_ref[...], wes[...]) + _dot(hd_ref[...], wed[...])
               + _dot(e_ref[...], wee[...]) + be[...])
    e20 = _dot(en, wep[...]) + bep[...]
    r = _dot(_relu(_dot(e20, wr1[...]) + br1[...]), wr2[...]) + br2[...]
    srcv = srcf_ref[0]
    ge = (srcv >= starts_ref[...]).astype(F32)
    gid = jnp.sum(ge, axis=1, keepdims=True)
    lane = jax.lax.broadcasted_iota(jnp.int32, srcv.shape[:1] + (G_GRAPHS,),
                                    1).astype(F32) + 1.0
    onehot = (gid == lane).astype(F32)
    gpart = jax.lax.dot_general(onehot, r, (((0,), (0,)), ((), ())),
                                preferred_element_type=F32)

    @pl.when(pl.program_id(0) == 0)
    def _():
        g_ref[...] = jnp.zeros_like(g_ref)

    g_ref[...] += gpart


def _edge_last(gat, e, wes, wed, wee, be, wep, bep, wr1, br1, wr2, br2,
               srcf, starts):
    e_edges = gat.shape[0] // 2
    dout = wr2.shape[1]
    eb = _pick_block(e_edges, _EDGE_CANDS)
    off = e_edges // eb
    row = lambda i: (i, 0)
    srcf3 = srcf.reshape(e_edges // eb, eb, 1)
    return pl.pallas_call(
        _edge_last_body,
        grid=(e_edges // eb,),
        in_specs=[
            pl.BlockSpec((eb, HP), row),
            pl.BlockSpec((eb, HP), lambda i: (i + off, 0)),
            pl.BlockSpec((eb, e.shape[1]), row),
            _full(wes.shape), _full(wed.shape), _full(wee.shape),
            _full(be.shape),
            _full(wep.shape), _full(bep.shape),
            _full(wr1.shape), _full(br1.shape),
            _full(wr2.shape), _full(br2.shape),
            pl.BlockSpec((1, eb, 1), lambda i: (i, 0, 0)),
            _full((1, G_GRAPHS)),
        ],
        out_specs=pl.BlockSpec((G_GRAPHS, dout), lambda i: (0, 0)),
        out_shape=jax.ShapeDtypeStruct((G_GRAPHS, dout), F32),
    )(gat, gat, e, wes, wed, wee, be, wep, bep, wr1, br1, wr2, br2,
      srcf3, starts)


def _update_body(h_ref, a_ref, wuh, wua, bu, o_ref):
    agg = a_ref[0] + a_ref[1]
    o_ref[...] = _relu(_dot(h_ref[...], wuh[...]) + _dot(agg, wua[...])
                       + bu[...])


def _update(h, agg2, wuh, wua, bu):
    n = h.shape[0]
    hd = agg2.shape[2]
    nb = _pick_block(n, _NODE_CANDS)
    row = lambda i: (i, 0)
    return pl.pallas_call(
        _update_body,
        grid=(n // nb,),
        in_specs=[pl.BlockSpec((nb, HP), row),
                  pl.BlockSpec((2, nb, hd), lambda i: (0, i, 0)),
                  _full(wuh.shape), _full(wua.shape), _full(bu.shape)],
        out_specs=pl.BlockSpec((nb, HP), row),
        out_shape=jax.ShapeDtypeStruct((n, HP), F32),
    )(h, agg2, wuh, wua, bu)


_SUP = 512
_SUB = 128


def _sc_gather(h, idx_flat):
    n, d = h.shape
    m_idx = idx_flat.shape[0]
    assert m_idx % _SUP == 0
    nsup = m_idx // _SUP
    info = plsc.get_sparse_core_info()
    nw = info.num_cores * info.num_subcores
    nq = _SUP // _SUB
    mesh = plsc.VectorSubcoreMesh(core_axis_name="c", subcore_axis_name="s")

    @functools.partial(
        pl.kernel, mesh=mesh,
        out_type=jax.ShapeDtypeStruct((m_idx, d), F32),
        scratch_types=[
            pltpu.VMEM((nq, _SUB), jnp.int32),
            pltpu.VMEM((_SUP, d), F32),
            pltpu.SemaphoreType.DMA,
        ],
    )
    def gather_k(h_hbm, idx_hbm, out_hbm, idx_v, rows_v, sem):
        wid = lax.axis_index("s") * info.num_cores + lax.axis_index("c")
        ntrips = (nsup - wid + nw - 1) // nw

        def body(t, carry):
            base = (wid + t * nw) * _SUP
            for q in range(nq):
                pltpu.sync_copy(idx_hbm.at[pl.ds(base + q * _SUB, _SUB)],
                                idx_v.at[q])
            handles = [
                pltpu.async_copy(h_hbm.at[idx_v.at[q]],
                                 rows_v.at[pl.ds(q * _SUB, _SUB)], sem)
                for q in range(nq)
            ]
            for hnd in handles:
                hnd.wait()
            pltpu.sync_copy(rows_v, out_hbm.at[pl.ds(base, _SUP)])
            return carry

        lax.fori_loop(0, ntrips, body, 0)

    return gather_k(h, idx_flat)


def _sc_scatter(m, idx_flat, idx_offset, n):
    sup = 256
    e, d = m.shape
    assert e % sup == 0
    nsup = e // sup
    info = plsc.get_sparse_core_info()
    nw = info.num_cores * info.num_subcores
    nq = sup // _SUB
    nts = info.num_subcores
    rpt = ((n + nts - 1) // nts + 7) // 8 * 8
    n_pad = rpt * nts
    mesh = plsc.VectorSubcoreMesh(core_axis_name="c", subcore_axis_name="s")

    @functools.partial(
        pl.kernel, mesh=mesh,
        out_type=jax.ShapeDtypeStruct((info.num_cores, n_pad, d), F32),
        scratch_types=[
            pltpu.VMEM((nq, _SUB), jnp.int32),
            pltpu.VMEM((sup, d), F32),
            pltpu.VMEM_SHARED((n_pad, d), F32),
            pltpu.SemaphoreType.DMA,
        ],
    )
    def scatter_k(m_hbm, idx_hbm, z_hbm, out_hbm, idx_v, rows_v, acc_sh, sem):
        cid = lax.axis_index("c")
        sid = lax.axis_index("s")
        wid = sid * info.num_cores + cid
        pltpu.sync_copy(z_hbm, acc_sh.at[pl.ds(sid * rpt, rpt)])
        plsc.subcore_barrier()
        ntrips = (nsup - wid + nw - 1) // nw

        def body(t, carry):
            base = (wid + t * nw) * sup
            pltpu.sync_copy(m_hbm.at[pl.ds(base, sup)], rows_v)
            for q in range(nq):
                pltpu.sync_copy(idx_hbm.at[pl.ds(idx_offset + base + q * _SUB,
                                                 _SUB)], idx_v.at[q])
            for q in range(nq):
                pltpu.sync_copy(rows_v.at[pl.ds(q * _SUB, _SUB)],
                                acc_sh.at[idx_v.at[q]], add=True)
            return carry

        lax.fori_loop(0, ntrips, body, 0)
        plsc.subcore_barrier()
        pltpu.sync_copy(acc_sh.at[pl.ds(sid * rpt, rpt)],
                        out_hbm.at[cid, pl.ds(sid * rpt, rpt)])

    return scatter_k(m, idx_flat, jnp.zeros((rpt, d), F32))


def kernel(x, edge_attr, edge_index, batch, W_node, b_node, W_eenc, b_eenc,
           Wm, bm, Wm2, bm2, Wu, bu, We, be, Wep, bep, Wr1, br1, Wr2, br2):
    n = x.shape[0]
    e_edges = edge_index.shape[1]
    h_dim = W_node.shape[1]
    idx_flat = edge_index.reshape(2 * e_edges)
    batch_f = batch.astype(F32).reshape(n, 1)
    src_f = edge_index[0].astype(F32)

    wnode = _padc(W_node, HP)
    bnode = _padc(b_node.reshape(1, -1), HP)
    wms = [_padr(Wm[l][:h_dim], HP) for l in range(3)]
    wmd = [_padr(Wm[l][h_dim:2 * h_dim], HP) for l in range(3)]
    wme = [Wm[l][2 * h_dim:] for l in range(3)]
    wes = [_padr(We[l][:h_dim], HP) for l in range(3)]
    wed = [_padr(We[l][h_dim:2 * h_dim], HP) for l in range(3)]
    wee = [We[l][2 * h_dim:] for l in range(3)]
    wuh = [_padc(_padr(Wu[l][:h_dim], HP), HP) for l in range(2)]
    wua = [_padc(_padr(Wu[l][h_dim:], HP), HP) for l in range(2)]
    bup = [_padc(bu[l].reshape(1, -1), HP) for l in range(2)]
    bm_ = [bm[l].reshape(1, -1) for l in range(3)]
    wm2 = [_padc(Wm2[l], HP) for l in range(2)]
    bm2_ = [_padc(bm2[l].reshape(1, -1), HP) for l in range(2)]
    be_ = [be[l].reshape(1, -1) for l in range(3)]

    h, starts = _enc_node(x, wnode, bnode, batch_f)

    gat = _sc_gather(h, idx_flat)
    m, e = _edge_layer(gat, edge_attr, W_eenc, b_eenc.reshape(1, -1),
                       wms[0], wmd[0], wme[0], bm_[0], wm2[0], bm2_[0],
                       wes[0], wed[0], wee[0], be_[0])
    agg2 = _sc_scatter(m, idx_flat, e_edges, n)
    h = _update(h, agg2, wuh[0], wua[0], bup[0])

    gat = _sc_gather(h, idx_flat)
    m, e = _edge_layer(gat, e, None, None,
                       wms[1], wmd[1], wme[1], bm_[1], wm2[1], bm2_[1],
                       wes[1], wed[1], wee[1], be_[1])
    agg2 = _sc_scatter(m, idx_flat, e_edges, n)
    h = _update(h, agg2, wuh[1], wua[1], bup[1])

    gat = _sc_gather(h, idx_flat)
    g = _edge_last(gat, e, wes[2], wed[2], wee[2], be_[2], Wep,
                   bep.reshape(1, -1), Wr1, br1.reshape(1, -1), Wr2,
                   br2.reshape(1, -1), src_f, starts)
    return g


# speedup vs baseline: 3.8364x; 1.0866x over previous
import functools

import jax
import jax.numpy as jnp
from jax import lax
from jax.experimental import pallas as pl
from jax.experimental.pallas import tpu as pltpu
from jax.experimental.pallas import tpu_sc as plsc

F32 = jnp.float32
G_GRAPHS = 128
HP = 128


def _relu(v):
    return jnp.maximum(v, 0.0)


def _dot(a, b):
    return jax.lax.dot_general(a, b, (((1,), (0,)), ((), ())),
                               preferred_element_type=F32)


def _pick_block(total, cands):
    for c in cands:
        if total % c == 0:
            return c
    return total


def _padc(w, cols):
    return jnp.pad(w, ((0, 0), (0, cols - w.shape[1])))


def _padr(w, rows):
    return jnp.pad(w, ((0, rows - w.shape[0]), (0, 0)))


def _full(shape):
    return pl.BlockSpec(shape, lambda i: tuple(0 for _ in shape))


_EDGE_CANDS = [3200, 2560, 2048, 1600, 1280, 1024, 800, 640, 512, 400, 320,
               256, 200, 160, 128, 80, 64]
_NODE_CANDS = [2000, 1000, 500, 200, 100, 50, 10]


def _enc_node_body(x_ref, w_ref, b_ref, batch_ref, h_ref, starts_ref):
    h_ref[...] = _relu(_dot(x_ref[...], w_ref[...]) + b_ref[...])
    lane = jax.lax.broadcasted_iota(jnp.int32, (1, G_GRAPHS), 1).astype(F32)
    lt = (batch_ref[...] < lane).astype(F32)
    part = jnp.sum(lt, axis=0, keepdims=True)

    @pl.when(pl.program_id(0) == 0)
    def _():
        starts_ref[...] = jnp.zeros_like(starts_ref)

    starts_ref[...] += part


def _enc_node(x, w, b, batch_f):
    n, din = x.shape
    nb = _pick_block(n, _NODE_CANDS)
    return pl.pallas_call(
        _enc_node_body,
        grid=(n // nb,),
        in_specs=[
            pl.BlockSpec((nb, din), lambda i: (i, 0)),
            pl.BlockSpec((din, HP), lambda i: (0, 0)),
            pl.BlockSpec((1, HP), lambda i: (0, 0)),
            pl.BlockSpec((nb, 1), lambda i: (i, 0)),
        ],
        out_specs=[
            pl.BlockSpec((nb, HP), lambda i: (i, 0)),
            pl.BlockSpec((1, G_GRAPHS), lambda i: (0, 0)),
        ],
        out_shape=[
            jax.ShapeDtypeStruct((n, HP), F32),
            jax.ShapeDtypeStruct((1, G_GRAPHS), F32),
        ],
    )(x, w, b, batch_f)


def _edge_layer_body(hs_ref, hd_ref, e_ref, wms, wmd, wme, bm, wm2, bm2,
                     wes, wed, wee, be, m_ref, en_ref, *, enc=None):
    hs = hs_ref[...]
    hd = hd_ref[...]
    e = e_ref[...]
    if enc is not None:
        wenc, benc = enc
        e = _relu(_dot(e, wenc[...]) + benc[...])
    mp = _relu(_dot(hs, wms[...]) + _dot(hd, wmd[...]) + _dot(e, wme[...])
               + bm[...])
    m_ref[...] = _dot(mp, wm2[...]) + bm2[...]
    en_ref[...] = _relu(_dot(hs, wes[...]) + _dot(hd, wed[...])
                        + _dot(e, wee[...]) + be[...])


def _edge_layer0_body(hs_ref, hd_ref, ea_ref, wenc, benc, wms, wmd, wme, bm,
                      wm2, bm2, wes, wed, wee, be, m_ref, en_ref):
    _edge_layer_body(hs_ref, hd_ref, ea_ref, wms, wmd, wme, bm, wm2, bm2,
                     wes, wed, wee, be, m_ref, en_ref, enc=(wenc, benc))


def _edge_layer(gat, e, wenc, benc, wms, wmd, wme, bm, wm2, bm2,
                wes, wed, wee, be):
    e_edges = gat.shape[0] // 2
    m_dim = wm2.shape[1]
    en_dim = wee.shape[1]
    de = e.shape[1]
    eb = _pick_block(e_edges, _EDGE_CANDS)
    off = e_edges // eb
    row = lambda i: (i, 0)
    in_specs = [pl.BlockSpec((eb, HP), row),
                pl.BlockSpec((eb, HP), lambda i: (i + off, 0)),
                pl.BlockSpec((eb, de), row)]
    args = [gat, gat, e]
    if wenc is not None:
        body = _edge_layer0_body
        in_specs += [_full(wenc.shape), _full(benc.shape)]
        args += [wenc, benc]
    else:
        body = _edge_layer_body
    for wref in (wms, wmd, wme, bm, wm2, bm2, wes, wed, wee, be):
        in_specs.append(_full(wref.shape))
        args.append(wref)
    return pl.pallas_call(
        body,
        grid=(e_edges // eb,),
        in_specs=in_specs,
        out_specs=[pl.BlockSpec((eb, m_dim), row),
                   pl.BlockSpec((eb, en_dim), row)],
        out_shape=[jax.ShapeDtypeStruct((e_edges, m_dim), F32),
                   jax.ShapeDtypeStruct((e_edges, en_dim), F32)],
    )(*args)


def _edge_last_body(hs_ref, hd_ref, e_ref, wes, wed, wee, be, wep, bep,
                    wr1, br1, wr2, br2, srcf_ref, starts_ref, g_ref):
    en = _relu(_dot(hs_ref[...], wes[...]) + _dot(hd_ref[...], wed[...])
               + _dot(e_ref[...], wee[...]) + be[...])
    e20 = _dot(en, wep[...]) + bep[...]
    r = _dot(_relu(_dot(e20, wr1[...]) + br1[...]), wr2[...]) + br2[...]
    srcv = srcf_ref[0]
    ge = (srcv >= starts_ref[...]).astype(F32)
    gid = jnp.sum(ge, axis=1, keepdims=True)
    lane = jax.lax.broadcasted_iota(jnp.int32, srcv.shape[:1] + (G_GRAPHS,),
                                    1).astype(F32) + 1.0
    onehot = (gid == lane).astype(F32)
    gpart = jax.lax.dot_general(onehot, r, (((0,), (0,)), ((), ())),
                                preferred_element_type=F32)

    @pl.when(pl.program_id(0) == 0)
    def _():
        g_ref[...] = jnp.zeros_like(g_ref)

    g_ref[...] += gpart


def _edge_last(gat, e, wes, wed, wee, be, wep, bep, wr1, br1, wr2, br2,
               srcf, starts):
    e_edges = gat.shape[0] // 2
    dout = wr2.shape[1]
    eb = _pick_block(e_edges, _EDGE_CANDS)
    off = e_edges // eb
    row = lambda i: (i, 0)
    srcf3 = srcf.reshape(e_edges // eb, eb, 1)
    return pl.pallas_call(
        _edge_last_body,
        grid=(e_edges // eb,),
        in_specs=[
            pl.BlockSpec((eb, HP), row),
            pl.BlockSpec((eb, HP), lambda i: (i + off, 0)),
            pl.BlockSpec((eb, e.shape[1]), row),
            _full(wes.shape), _full(wed.shape), _full(wee.shape),
            _full(be.shape),
            _full(wep.shape), _full(bep.shape),
            _full(wr1.shape), _full(br1.shape),
            _full(wr2.shape), _full(br2.shape),
            pl.BlockSpec((1, eb, 1), lambda i: (i, 0, 0)),
            _full((1, G_GRAPHS)),
        ],
        out_specs=pl.BlockSpec((G_GRAPHS, dout), lambda i: (0, 0)),
        out_shape=jax.ShapeDtypeStruct((G_GRAPHS, dout), F32),
    )(gat, gat, e, wes, wed, wee, be, wep, bep, wr1, br1, wr2, br2,
      srcf3, starts)


def _update_body(h_ref, a_ref, wuh, wua, bu, o_ref):
    agg = a_ref[0] + a_ref[1]
    o_ref[...] = _relu(_dot(h_ref[...], wuh[...]) + _dot(agg, wua[...])
                       + bu[...])


def _update(h, agg2, wuh, wua, bu):
    n = h.shape[0]
    hd = agg2.shape[2]
    nb = _pick_block(n, _NODE_CANDS)
    row = lambda i: (i, 0)
    return pl.pallas_call(
        _update_body,
        grid=(n // nb,),
        in_specs=[pl.BlockSpec((nb, HP), row),
                  pl.BlockSpec((2, nb, hd), lambda i: (0, i, 0)),
                  _full(wuh.shape), _full(wua.shape), _full(bu.shape)],
        out_specs=pl.BlockSpec((nb, HP), row),
        out_shape=jax.ShapeDtypeStruct((n, HP), F32),
    )(h, agg2, wuh, wua, bu)


_SUP = 256
_SUB = 128


def _sc_gather(h, idx_flat):
    n, d = h.shape
    m_idx = idx_flat.shape[0]
    assert m_idx % _SUP == 0
    nsup = m_idx // _SUP
    info = plsc.get_sparse_core_info()
    nw = info.num_cores * info.num_subcores
    nq = _SUP // _SUB
    mesh = plsc.VectorSubcoreMesh(core_axis_name="c", subcore_axis_name="s")

    @functools.partial(
        pl.kernel, mesh=mesh,
        out_type=jax.ShapeDtypeStruct((m_idx, d), F32),
        scratch_types=[
            pltpu.VMEM((2 * nq, _SUB), jnp.int32),
            pltpu.VMEM((2 * _SUP, d), F32),
            pltpu.SemaphoreType.DMA,
            pltpu.SemaphoreType.DMA,
        ],
    )
    def gather_k(h_hbm, idx_hbm, out_hbm, idx_v, rows_v, sem_a, sem_b):
        wid = lax.axis_index("s") * info.num_cores + lax.axis_index("c")
        ntrips = (nsup - wid + nw - 1) // nw
        npairs = ntrips // 2

        def fire(base, half, sem):
            for q in range(nq):
                pltpu.sync_copy(idx_hbm.at[pl.ds(base + q * _SUB, _SUB)],
                                idx_v.at[half * nq + q])
            return [
                pltpu.async_copy(
                    h_hbm.at[idx_v.at[half * nq + q]],
                    rows_v.at[pl.ds(half * _SUP + q * _SUB, _SUB)], sem)
                for q in range(nq)
            ]

        def drain(base, half, handles):
            for hnd in handles:
                hnd.wait()
            pltpu.sync_copy(rows_v.at[pl.ds(half * _SUP, _SUP)],
                            out_hbm.at[pl.ds(base, _SUP)])

        def body(t, carry):
            base_a = (wid + (2 * t) * nw) * _SUP
            base_b = (wid + (2 * t + 1) * nw) * _SUP
            ha = fire(base_a, 0, sem_a)
            hb = fire(base_b, 1, sem_b)
            drain(base_a, 0, ha)
            drain(base_b, 1, hb)
            return carry

        lax.fori_loop(0, npairs, body, 0)

        @pl.when(ntrips % 2 == 1)
        def _():
            base = (wid + (ntrips - 1) * nw) * _SUP
            drain(base, 0, fire(base, 0, sem_a))

    return gather_k(h, idx_flat)


def _sc_scatter(m, idx_flat, idx_offset, n):
    sup = 128
    e, d = m.shape
    assert e % sup == 0
    nsup = e // sup
    info = plsc.get_sparse_core_info()
    nw = info.num_cores * info.num_subcores
    nts = info.num_subcores
    rpt = ((n + nts - 1) // nts + 7) // 8 * 8
    n_pad = rpt * nts
    mesh = plsc.VectorSubcoreMesh(core_axis_name="c", subcore_axis_name="s")

    @functools.partial(
        pl.kernel, mesh=mesh,
        out_type=jax.ShapeDtypeStruct((info.num_cores, n_pad, d), F32),
        scratch_types=[
            pltpu.VMEM((2, _SUB), jnp.int32),
            pltpu.VMEM((2 * sup, d), F32),
            pltpu.VMEM_SHARED((n_pad, d), F32),
            pltpu.SemaphoreType.DMA,
            pltpu.SemaphoreType.DMA,
        ],
    )
    def scatter_k(m_hbm, idx_hbm, z_hbm, out_hbm, idx_v, rows_v, acc_sh,
                  sem_a, sem_b):
        cid = lax.axis_index("c")
        sid = lax.axis_index("s")
        wid = sid * info.num_cores + cid
        pltpu.sync_copy(z_hbm, acc_sh.at[pl.ds(sid * rpt, rpt)])
        plsc.subcore_barrier()
        ntrips = (nsup - wid + nw - 1) // nw
        npairs = ntrips // 2

        def fire(base, half, sem):
            return [
                pltpu.async_copy(m_hbm.at[pl.ds(base, sup)],
                                 rows_v.at[pl.ds(half * sup, sup)], sem),
                pltpu.async_copy(idx_hbm.at[pl.ds(idx_offset + base, sup)],
                                 idx_v.at[half], sem),
            ]

        def drain(half, handles):
            for hnd in handles:
                hnd.wait()
            pltpu.sync_copy(rows_v.at[pl.ds(half * sup, sup)],
                            acc_sh.at[idx_v.at[half]], add=True)

        def body(t, carry):
            base_a = (wid + (2 * t) * nw) * sup
            base_b = (wid + (2 * t + 1) * nw) * sup
            ha = fire(base_a, 0, sem_a)
            hb = fire(base_b, 1, sem_b)
            drain(0, ha)
            drain(1, hb)
            return carry

        lax.fori_loop(0, npairs, body, 0)

        @pl.when(ntrips % 2 == 1)
        def _():
            base = (wid + (ntrips - 1) * nw) * sup
            drain(0, fire(base, 0, sem_a))

        plsc.subcore_barrier()
        pltpu.sync_copy(acc_sh.at[pl.ds(sid * rpt, rpt)],
                        out_hbm.at[cid, pl.ds(sid * rpt, rpt)])

    return scatter_k(m, idx_flat, jnp.zeros((rpt, d), F32))


def kernel(x, edge_attr, edge_index, batch, W_node, b_node, W_eenc, b_eenc,
           Wm, bm, Wm2, bm2, Wu, bu, We, be, Wep, bep, Wr1, br1, Wr2, br2):
    n = x.shape[0]
    e_edges = edge_index.shape[1]
    h_dim = W_node.shape[1]
    idx_flat = edge_index.reshape(2 * e_edges)
    batch_f = batch.astype(F32).reshape(n, 1)
    src_f = edge_index[0].astype(F32)

    wnode = _padc(W_node, HP)
    bnode = _padc(b_node.reshape(1, -1), HP)
    wms = [_padr(Wm[l][:h_dim], HP) for l in range(3)]
    wmd = [_padr(Wm[l][h_dim:2 * h_dim], HP) for l in range(3)]
    wme = [Wm[l][2 * h_dim:] for l in range(3)]
    wes = [_padr(We[l][:h_dim], HP) for l in range(3)]
    wed = [_padr(We[l][h_dim:2 * h_dim], HP) for l in range(3)]
    wee = [We[l][2 * h_dim:] for l in range(3)]
    wuh = [_padc(_padr(Wu[l][:h_dim], HP), HP) for l in range(2)]
    wua = [_padc(_padr(Wu[l][h_dim:], HP), HP) for l in range(2)]
    bup = [_padc(bu[l].reshape(1, -1), HP) for l in range(2)]
    bm_ = [bm[l].reshape(1, -1) for l in range(3)]
    wm2 = [_padc(Wm2[l], HP) for l in range(2)]
    bm2_ = [_padc(bm2[l].reshape(1, -1), HP) for l in range(2)]
    be_ = [be[l].reshape(1, -1) for l in range(3)]

    h, starts = _enc_node(x, wnode, bnode, batch_f)

    gat = _sc_gather(h, idx_flat)
    m, e = _edge_layer(gat, edge_attr, W_eenc, b_eenc.reshape(1, -1),
                       wms[0], wmd[0], wme[0], bm_[0], wm2[0], bm2_[0],
                       wes[0], wed[0], wee[0], be_[0])
    agg2 = _sc_scatter(m, idx_flat, e_edges, n)
    h = _update(h, agg2, wuh[0], wua[0], bup[0])

    gat = _sc_gather(h, idx_flat)
    m, e = _edge_layer(gat, e, None, None,
                       wms[1], wmd[1], wme[1], bm_[1], wm2[1], bm2_[1],
                       wes[1], wed[1], wee[1], be_[1])
    agg2 = _sc_scatter(m, idx_flat, e_edges, n)
    h = _update(h, agg2, wuh[1], wua[1], bup[1])

    gat = _sc_gather(h, idx_flat)
    g = _edge_last(gat, e, wes[2], wed[2], wee[2], be_[2], Wep,
                   bep.reshape(1, -1), Wr1, br1.reshape(1, -1), Wr2,
                   br2.reshape(1, -1), src_f, starts)
    return g


# speedup vs baseline: 3.8424x; 1.0016x over previous
import functools

import jax
import jax.numpy as jnp
from jax import lax
from jax.experimental import pallas as pl
from jax.experimental.pallas import tpu as pltpu
from jax.experimental.pallas import tpu_sc as plsc

F32 = jnp.float32
G_GRAPHS = 128
HP = 128


def _relu(v):
    return jnp.maximum(v, 0.0)


def _dot(a, b):
    return jax.lax.dot_general(a, b, (((1,), (0,)), ((), ())),
                               preferred_element_type=F32)


def _pick_block(total, cands):
    for c in cands:
        if total % c == 0:
            return c
    return total


def _padc(w, cols):
    return jnp.pad(w, ((0, 0), (0, cols - w.shape[1])))


def _padr(w, rows):
    return jnp.pad(w, ((0, rows - w.shape[0]), (0, 0)))


def _full(shape):
    return pl.BlockSpec(shape, lambda i: tuple(0 for _ in shape))


_EDGE_CANDS = [3200, 2560, 2048, 1600, 1280, 1024, 800, 640, 512, 400, 320,
               256, 200, 160, 128, 80, 64]
_NODE_CANDS = [2000, 1000, 500, 200, 100, 50, 10]


def _enc_node_body(x_ref, w_ref, b_ref, batch_ref, h_ref, starts_ref):
    h_ref[...] = _relu(_dot(x_ref[...], w_ref[...]) + b_ref[...])
    lane = jax.lax.broadcasted_iota(jnp.int32, (1, G_GRAPHS), 1).astype(F32)
    lt = (batch_ref[...] < lane).astype(F32)
    part = jnp.sum(lt, axis=0, keepdims=True)

    @pl.when(pl.program_id(0) == 0)
    def _():
        starts_ref[...] = jnp.zeros_like(starts_ref)

    starts_ref[...] += part


def _enc_node(x, w, b, batch_f):
    n, din = x.shape
    nb = _pick_block(n, _NODE_CANDS)
    return pl.pallas_call(
        _enc_node_body,
        grid=(n // nb,),
        in_specs=[
            pl.BlockSpec((nb, din), lambda i: (i, 0)),
            pl.BlockSpec((din, HP), lambda i: (0, 0)),
            pl.BlockSpec((1, HP), lambda i: (0, 0)),
            pl.BlockSpec((nb, 1), lambda i: (i, 0)),
        ],
        out_specs=[
            pl.BlockSpec((nb, HP), lambda i: (i, 0)),
            pl.BlockSpec((1, G_GRAPHS), lambda i: (0, 0)),
        ],
        out_shape=[
            jax.ShapeDtypeStruct((n, HP), F32),
            jax.ShapeDtypeStruct((1, G_GRAPHS), F32),
        ],
    )(x, w, b, batch_f)


def _edge_layer_body(hs_ref, hd_ref, e_ref, wms, wmd, wme, bm, wm2, bm2,
                     wes, wed, wee, be, m_ref, en_ref, *, enc=None):
    hs = hs_ref[...]
    hd = hd_ref[...]
    e = e_ref[...]
    if enc is not None:
        wenc, benc = enc
        e = _relu(_dot(e, wenc[...]) + benc[...])
    mp = _relu(_dot(hs, wms[...]) + _dot(hd, wmd[...]) + _dot(e, wme[...])
               + bm[...])
    m_ref[...] = _dot(mp, wm2[...]) + bm2[...]
    en_ref[...] = _relu(_dot(hs, wes[...]) + _dot(hd, wed[...])
                        + _dot(e, wee[...]) + be[...])


def _edge_layer0_body(hs_ref, hd_ref, ea_ref, wenc, benc, wms, wmd, wme, bm,
                      wm2, bm2, wes, wed, wee, be, m_ref, en_ref):
    _edge_layer_body(hs_ref, hd_ref, ea_ref, wms, wmd, wme, bm, wm2, bm2,
                     wes, wed, wee, be, m_ref, en_ref, enc=(wenc, benc))


def _edge_layer(gat, e, e_off, wenc, benc, wms, wmd, wme, bm, wm2, bm2,
                wes, wed, wee, be):
    e_edges = gat.shape[0] // 2
    m_dim = wm2.shape[1]
    en_dim = wee.shape[1]
    de = e.shape[1]
    eb = _pick_block(e_edges, _EDGE_CANDS)
    off = e_edges // eb
    row = lambda i: (i, 0)
    in_specs = [pl.BlockSpec((eb, HP), row),
                pl.BlockSpec((eb, HP), lambda i: (i + off, 0)),
                pl.BlockSpec((eb, de), lambda i: (i + e_off, 0))]
    args = [gat, gat, e]
    if wenc is not None:
        body = _edge_layer0_body
        in_specs += [_full(wenc.shape), _full(benc.shape)]
        args += [wenc, benc]
    else:
        body = _edge_layer_body
    for wref in (wms, wmd, wme, bm, wm2, bm2, wes, wed, wee, be):
        in_specs.append(_full(wref.shape))
        args.append(wref)
    return pl.pallas_call(
        body,
        grid=(e_edges // eb,),
        in_specs=in_specs,
        out_specs=[pl.BlockSpec((eb, m_dim), row),
                   pl.BlockSpec((eb, en_dim), row)],
        out_shape=[jax.ShapeDtypeStruct((e_edges, m_dim), F32),
                   jax.ShapeDtypeStruct((e_edges, en_dim), F32)],
    )(*args)


def _edge_last_body(hs_ref, hd_ref, e_ref, wes, wed, wee, be, wep, bep,
                    wr1, br1, wr2, br2, srcf_ref, starts_ref, g_ref):
    en = _relu(_dot(hs_ref[...], wes[...]) + _dot(hd_ref[...], wed[...])
               + _dot(e_ref[...], wee[...]) + be[...])
    e20 = _dot(en, wep[...]) + bep[...]
    r = _dot(_relu(_dot(e20, wr1[...]) + br1[...]), wr2[...]) + br2[...]
    srcv = srcf_ref[0]
    ge = (srcv >= starts_ref[...]).astype(F32)
    gid = jnp.sum(ge, axis=1, keepdims=True)
    lane = jax.lax.broadcasted_iota(jnp.int32, srcv.shape[:1] + (G_GRAPHS,),
                                    1).astype(F32) + 1.0
    onehot = (gid == lane).astype(F32)
    gpart = jax.lax.dot_general(onehot, r, (((0,), (0,)), ((), ())),
                                preferred_element_type=F32)

    @pl.when(pl.program_id(0) == 0)
    def _():
        g_ref[...] = jnp.zeros_like(g_ref)

    g_ref[...] += gpart


def _edge_last(gat, e, e_off, wes, wed, wee, be, wep, bep, wr1, br1, wr2, br2,
               srcf, starts):
    e_edges = gat.shape[0] // 2
    dout = wr2.shape[1]
    eb = _pick_block(e_edges, _EDGE_CANDS)
    off = e_edges // eb
    row = lambda i: (i, 0)
    srcf3 = srcf.reshape(e_edges // eb, eb, 1)
    return pl.pallas_call(
        _edge_last_body,
        grid=(e_edges // eb,),
        in_specs=[
            pl.BlockSpec((eb, HP), row),
            pl.BlockSpec((eb, HP), lambda i: (i + off, 0)),
            pl.BlockSpec((eb, e.shape[1]), lambda i: (i + e_off, 0)),
            _full(wes.shape), _full(wed.shape), _full(wee.shape),
            _full(be.shape),
            _full(wep.shape), _full(bep.shape),
            _full(wr1.shape), _full(br1.shape),
            _full(wr2.shape), _full(br2.shape),
            pl.BlockSpec((1, eb, 1), lambda i: (i, 0, 0)),
            _full((1, G_GRAPHS)),
        ],
        out_specs=pl.BlockSpec((G_GRAPHS, dout), lambda i: (0, 0)),
        out_shape=jax.ShapeDtypeStruct((G_GRAPHS, dout), F32),
    )(gat, gat, e, wes, wed, wee, be, wep, bep, wr1, br1, wr2, br2,
      srcf3, starts)


def _update_body(h_ref, a_ref, b_ref, wuh, wua, bu, o_ref):
    agg = a_ref[0] + a_ref[1] + b_ref[0] + b_ref[1]
    o_ref[...] = _relu(_dot(h_ref[...], wuh[...]) + _dot(agg, wua[...])
                       + bu[...])


def _update(h, agg_a, agg_b, wuh, wua, bu):
    n = h.shape[0]
    hd = agg_a.shape[2]
    nb = _pick_block(n, _NODE_CANDS)
    row = lambda i: (i, 0)
    return pl.pallas_call(
        _update_body,
        grid=(n // nb,),
        in_specs=[pl.BlockSpec((nb, HP), row),
                  pl.BlockSpec((2, nb, hd), lambda i: (0, i, 0)),
                  pl.BlockSpec((2, nb, hd), lambda i: (0, i, 0)),
                  _full(wuh.shape), _full(wua.shape), _full(bu.shape)],
        out_specs=pl.BlockSpec((nb, HP), row),
        out_shape=jax.ShapeDtypeStruct((n, HP), F32),
    )(h, agg_a, agg_b, wuh, wua, bu)


_SUP = 256
_SUB = 128


def _sc_gather(h, idx_flat):
    n, d = h.shape
    m_idx = idx_flat.shape[0]
    assert m_idx % _SUP == 0
    nsup = m_idx // _SUP
    info = plsc.get_sparse_core_info()
    nw = info.num_cores * info.num_subcores
    nq = _SUP // _SUB
    mesh = plsc.VectorSubcoreMesh(core_axis_name="c", subcore_axis_name="s")

    @functools.partial(
        pl.kernel, mesh=mesh,
        out_type=jax.ShapeDtypeStruct((m_idx, d), F32),
        scratch_types=[
            pltpu.VMEM((2 * nq, _SUB), jnp.int32),
            pltpu.VMEM((2 * _SUP, d), F32),
            pltpu.SemaphoreType.DMA,
            pltpu.SemaphoreType.DMA,
        ],
    )
    def gather_k(h_hbm, idx_hbm, out_hbm, idx_v, rows_v, sem_a, sem_b):
        wid = lax.axis_index("s") * info.num_cores + lax.axis_index("c")
        ntrips = (nsup - wid + nw - 1) // nw
        npairs = ntrips // 2

        def fire(base, half, sem):
            for q in range(nq):
                pltpu.sync_copy(idx_hbm.at[pl.ds(base + q * _SUB, _SUB)],
                                idx_v.at[half * nq + q])
            return [
                pltpu.async_copy(
                    h_hbm.at[idx_v.at[half * nq + q]],
                    rows_v.at[pl.ds(half * _SUP + q * _SUB, _SUB)], sem)
                for q in range(nq)
            ]

        def drain(base, half, handles):
            for hnd in handles:
                hnd.wait()
            pltpu.sync_copy(rows_v.at[pl.ds(half * _SUP, _SUP)],
                            out_hbm.at[pl.ds(base, _SUP)])

        def body(t, carry):
            base_a = (wid + (2 * t) * nw) * _SUP
            base_b = (wid + (2 * t + 1) * nw) * _SUP
            ha = fire(base_a, 0, sem_a)
            hb = fire(base_b, 1, sem_b)
            drain(base_a, 0, ha)
            drain(base_b, 1, hb)
            return carry

        lax.fori_loop(0, npairs, body, 0)

        @pl.when(ntrips % 2 == 1)
        def _():
            base = (wid + (ntrips - 1) * nw) * _SUP
            drain(base, 0, fire(base, 0, sem_a))

    return gather_k(h, idx_flat)


def _sc_scatter(m, idx_flat, idx_offset, n):
    sup = 128
    e, d = m.shape
    assert e % sup == 0
    nsup = e // sup
    info = plsc.get_sparse_core_info()
    nw = info.num_cores * info.num_subcores
    nts = info.num_subcores
    rpt = ((n + nts - 1) // nts + 7) // 8 * 8
    n_pad = rpt * nts
    mesh = plsc.VectorSubcoreMesh(core_axis_name="c", subcore_axis_name="s")

    @functools.partial(
        pl.kernel, mesh=mesh,
        out_type=jax.ShapeDtypeStruct((info.num_cores, n_pad, d), F32),
        scratch_types=[
            pltpu.VMEM((2, _SUB), jnp.int32),
            pltpu.VMEM((2 * sup, d), F32),
            pltpu.VMEM_SHARED((n_pad, d), F32),
            pltpu.SemaphoreType.DMA,
            pltpu.SemaphoreType.DMA,
        ],
    )
    def scatter_k(m_hbm, idx_hbm, z_hbm, out_hbm, idx_v, rows_v, acc_sh,
                  sem_a, sem_b):
        cid = lax.axis_index("c")
        sid = lax.axis_index("s")
        wid = sid * info.num_cores + cid
        pltpu.sync_copy(z_hbm, acc_sh.at[pl.ds(sid * rpt, rpt)])
        plsc.subcore_barrier()
        ntrips = (nsup - wid + nw - 1) // nw
        npairs = ntrips // 2

        def fire(base, half, sem):
            return [
                pltpu.async_copy(m_hbm.at[pl.ds(base, sup)],
                                 rows_v.at[pl.ds(half * sup, sup)], sem),
                pltpu.async_copy(idx_hbm.at[pl.ds(idx_offset + base, sup)],
                                 idx_v.at[half], sem),
            ]

        def drain(half, handles):
            for hnd in handles:
                hnd.wait()
            pltpu.sync_copy(rows_v.at[pl.ds(half * sup, sup)],
                            acc_sh.at[idx_v.at[half]], add=True)

        def body(t, carry):
            base_a = (wid + (2 * t) * nw) * sup
            base_b = (wid + (2 * t + 1) * nw) * sup
            ha = fire(base_a, 0, sem_a)
            hb = fire(base_b, 1, sem_b)
            drain(0, ha)
            drain(1, hb)
            return carry

        lax.fori_loop(0, npairs, body, 0)

        @pl.when(ntrips % 2 == 1)
        def _():
            base = (wid + (ntrips - 1) * nw) * sup
            drain(0, fire(base, 0, sem_a))

        plsc.subcore_barrier()
        pltpu.sync_copy(acc_sh.at[pl.ds(sid * rpt, rpt)],
                        out_hbm.at[cid, pl.ds(sid * rpt, rpt)])

    return scatter_k(m, idx_flat, jnp.zeros((rpt, d), F32))


def kernel(x, edge_attr, edge_index, batch, W_node, b_node, W_eenc, b_eenc,
           Wm, bm, Wm2, bm2, Wu, bu, We, be, Wep, bep, Wr1, br1, Wr2, br2):
    n = x.shape[0]
    e_edges = edge_index.shape[1]
    e2 = e_edges // 2
    h_dim = W_node.shape[1]
    src = edge_index[0]
    dst = edge_index[1]
    idx_half = [jnp.concatenate([src[:e2], dst[:e2]]),
                jnp.concatenate([src[e2:], dst[e2:]])]
    batch_f = batch.astype(F32).reshape(n, 1)
    src_f = [src[:e2].astype(F32), src[e2:].astype(F32)]

    wnode = _padc(W_node, HP)
    bnode = _padc(b_node.reshape(1, -1), HP)
    wms = [_padr(Wm[l][:h_dim], HP) for l in range(3)]
    wmd = [_padr(Wm[l][h_dim:2 * h_dim], HP) for l in range(3)]
    wme = [Wm[l][2 * h_dim:] for l in range(3)]
    wes = [_padr(We[l][:h_dim], HP) for l in range(3)]
    wed = [_padr(We[l][h_dim:2 * h_dim], HP) for l in range(3)]
    wee = [We[l][2 * h_dim:] for l in range(3)]
    wuh = [_padc(_padr(Wu[l][:h_dim], HP), HP) for l in range(2)]
    wua = [_padc(_padr(Wu[l][h_dim:], HP), HP) for l in range(2)]
    bup = [_padc(bu[l].reshape(1, -1), HP) for l in range(2)]
    bm_ = [bm[l].reshape(1, -1) for l in range(3)]
    wm2 = [_padc(Wm2[l], HP) for l in range(2)]
    bm2_ = [_padc(bm2[l].reshape(1, -1), HP) for l in range(2)]
    be_ = [be[l].reshape(1, -1) for l in range(3)]

    h, starts = _enc_node(x, wnode, bnode, batch_f)

    e_half = [edge_attr, edge_attr]
    e_offs = [0, e2 // _pick_block(e2, _EDGE_CANDS)]
    enc_w = [W_eenc, b_eenc.reshape(1, -1)]

    for l in range(2):
        gats = [_sc_gather(h, idx_half[0]), _sc_gather(h, idx_half[1])]
        outs = []
        for p in range(2):
            outs.append(_edge_layer(
                gats[p], e_half[p], e_offs[p], enc_w[0], enc_w[1],
                wms[l], wmd[l], wme[l], bm_[l], wm2[l], bm2_[l],
                wes[l], wed[l], wee[l], be_[l]))
        aggs = [_sc_scatter(outs[p][0], idx_half[p], e2, n) for p in range(2)]
        h = _update(h, aggs[0], aggs[1], wuh[l], wua[l], bup[l])
        e_half = [outs[0][1], outs[1][1]]
        e_offs = [0, 0]
        enc_w = [None, None]

    gats = [_sc_gather(h, idx_half[0]), _sc_gather(h, idx_half[1])]
    g = None
    for p in range(2):
        gp = _edge_last(gats[p], e_half[p], e_offs[p], wes[2], wed[2],
                        wee[2], be_[2], Wep, bep.reshape(1, -1), Wr1,
                        br1.reshape(1, -1), Wr2, br2.reshape(1, -1),
                        src_f[p], starts)
        g = gp if g is None else g + gp
    return g
